# baseline (speedup 1.0000x reference)
import functools

import jax
import jax.numpy as jnp
from jax import lax
from jax.experimental import pallas as pl
from jax.experimental.pallas import tpu as pltpu
from jax.experimental.pallas import tpu_sc as plsc

_NW = 32
_NC = 2

_LEVELS = [
    (10000, 16384, 128, 7000, 224, 7168, 56),
    (7000, 8192, 64, 3500, 112, 3584, 28),
    (3500, 4096, 32, 1050, 48, 1536, 12),
]


def _iota_e(R):
    return (lax.broadcasted_iota(jnp.int32, (R, 128), 0) * 128 +
            lax.broadcasted_iota(jnp.int32, (R, 128), 1))


def _bitonic(key, idx, P, R):
    e = _iota_e(R)
    LOG = P.bit_length() - 1
    for jj in range(1, LOG + 1):
        size = 1 << jj
        for kk in [1 << t for t in range(jj - 1, -1, -1)]:
            if kk >= 128:
                m = kk // 128
                def rowswap(a, m=m):
                    a4 = a.reshape(R // (2 * m), 2, m, 128)
                    return jnp.concatenate([a4[:, 1], a4[:, 0]],
                                           axis=1).reshape(R, 128)
                pk, pi = rowswap(key), rowswap(idx)
            else:
                mb = (e & kk) != 0
                pk = jnp.where(mb, jnp.roll(key, kk, axis=1),
                               jnp.roll(key, -kk, axis=1))
                pi = jnp.where(mb, jnp.roll(idx, kk, axis=1),
                               jnp.roll(idx, -kk, axis=1))
            upper = (e & kk) != 0
            desc = (e & size) != 0
            p_less = (pk < key) | ((pk == key) & (pi < idx))
            take_p = p_less ^ upper ^ desc
            key = jnp.where(take_p, pk, key)
            idx = jnp.where(take_p, pi, idx)
    return key, idx


def _vreg_stages(k8, i8, v, jjs, LOG):
    lr = lax.broadcasted_iota(jnp.int32, (8, 128), 0)
    c = lax.broadcasted_iota(jnp.int32, (8, 128), 1)
    le = lr * 128 + c
    for jj in jjs:
        size = 1 << jj
        for kk in [1 << t for t in range(min(jj, 10) - 1, -1, -1)]:
            if kk >= 128:
                m = kk // 128
                mb = (lr & m) != 0
                pk = jnp.where(mb, jnp.roll(k8, m, axis=0),
                               jnp.roll(k8, -m, axis=0))
                pi = jnp.where(mb, jnp.roll(i8, m, axis=0),
                               jnp.roll(i8, -m, axis=0))
            else:
                mb = (c & kk) != 0
                pk = jnp.where(mb, jnp.roll(k8, kk, axis=1),
                               jnp.roll(k8, -kk, axis=1))
                pi = jnp.where(mb, jnp.roll(i8, kk, axis=1),
                               jnp.roll(i8, -kk, axis=1))
            upper = (le & kk) != 0
            if size < 1024:
                desc = (le & size) != 0
            else:
                desc = jnp.full((8, 128), True) & (((v * 1024) & size) != 0)
            p_less = (pk < k8) | ((pk == k8) & (pi < i8))
            take_p = p_less ^ upper ^ desc
            k8 = jnp.where(take_p, pk, k8)
            i8 = jnp.where(take_p, pi, i8)
    return k8, i8


def _bitonic_ref(key_ref, idx_ref, P, R):
    LOG = P.bit_length() - 1
    R8 = R // 8

    def within_pass(jjs):
        def body(v, _):
            sl = pl.ds(v * 8, 8)
            k8, i8 = _vreg_stages(key_ref[sl, :], idx_ref[sl, :], v, jjs, LOG)
            key_ref[sl, :] = k8
            idx_ref[sl, :] = i8
            return 0
        lax.fori_loop(0, R8, body, 0)

    within_pass(list(range(1, min(10, LOG) + 1)))
    for jj in range(11, LOG + 1):
        size = 1 << jj
        for kk in [1 << t for t in range(jj - 1, 9, -1)]:
            m = kk // 1024

            def cross_body(t, _, m=m, size=size):
                v_lo = (t // m) * (2 * m) + t % m
                v_hi = v_lo + m
                slo = pl.ds(v_lo * 8, 8)
                shi = pl.ds(v_hi * 8, 8)
                k_lo, i_lo = key_ref[slo, :], idx_ref[slo, :]
                k_hi, i_hi = key_ref[shi, :], idx_ref[shi, :]
                desc = ((v_lo * 1024) & size) != 0
                hless = (k_hi < k_lo) | ((k_hi == k_lo) & (i_hi < i_lo))
                sel = hless ^ desc
                key_ref[slo, :] = jnp.where(sel, k_hi, k_lo)
                idx_ref[slo, :] = jnp.where(sel, i_hi, i_lo)
                key_ref[shi, :] = jnp.where(sel, k_lo, k_hi)
                idx_ref[shi, :] = jnp.where(sel, i_lo, i_hi)
                return 0
            lax.fori_loop(0, R8 // 2, cross_body, 0)
        within_pass([jj])


def _diag_extract(M, n, P, R):
    Mp = jnp.concatenate([M, jnp.zeros((P - n, 128), jnp.float32)], axis=0)
    eye = (lax.broadcasted_iota(jnp.int32, (128, 128), 0) ==
           lax.broadcasted_iota(jnp.int32, (128, 128), 1)
           ).astype(jnp.float32)
    return jnp.sum(Mp.reshape(R, 128, 128) * eye[None], axis=1)


def _keys_from_scores(sc, n, R):
    sc = sc + 0.0
    b = lax.bitcast_convert_type(sc, jnp.uint32)
    sign = b >> 31
    u_asc = jnp.where(sign == jnp.uint32(1), ~b, b | jnp.uint32(0x80000000))
    key = ~u_asc
    return jnp.where(_iota_e(R) < n, key, jnp.uint32(0xFFFFFFFF))


def _make_a_body(n, P, R):
    def body(cur_ref, w2_ref, dot_ref):
        M = jnp.dot(cur_ref[...], w2_ref[...],
                    preferred_element_type=jnp.float32)
        dot_ref[...] = _diag_extract(M, n, P, R)
    return body


def _a_level(cur, pw, n, P, R):
    W2 = jnp.broadcast_to(pw[:, None], (128, 128))
    return pl.pallas_call(
        _make_a_body(n, P, R),
        out_shape=jax.ShapeDtypeStruct((R, 128), jnp.float32),
    )(cur, W2)


def _make_b_body(n, P, R, kpR, with_next):
    def body(*refs):
        if with_next:
            (cur_ref, sfull_ref, slane_ref, wt_ref, b_ref, w2n_ref,
             z_ref, dn_ref, perm_ref, key_ref, idx_ref) = refs
        else:
            (cur_ref, sfull_ref, slane_ref, wt_ref, b_ref,
             z_ref, perm_ref, key_ref, idx_ref) = refs
        z = jnp.dot(cur_ref[...] * sfull_ref[...], wt_ref[...],
                    preferred_element_type=jnp.float32) + b_ref[...]
        z_ref[...] = z
        if with_next:
            Dn = jnp.dot(z, w2n_ref[...], preferred_element_type=jnp.float32)
            dn_ref[...] = _diag_extract(Dn, n, P, R)
        key_ref[...] = _keys_from_scores(slane_ref[...], n, R)
        idx_ref[...] = _iota_e(R)
        _bitonic_ref(key_ref, idx_ref, P, R)
        perm_ref[...] = idx_ref[:kpR, :]
    return body


def _b_level(cur, s_full, s_lane, W, b, pw_next, n, P, R, kpR):
    with_next = pw_next is not None
    outs = [jax.ShapeDtypeStruct((n, 128), jnp.float32)]
    if with_next:
        outs.append(jax.ShapeDtypeStruct((R, 128), jnp.float32))
    outs.append(jax.ShapeDtypeStruct((kpR, 128), jnp.int32))
    args = [cur, s_full, s_lane, W.T, b[None, :]]
    if with_next:
        args.append(jnp.broadcast_to(pw_next[:, None], (128, 128)))
    return pl.pallas_call(
        _make_b_body(n, P, R, kpR, with_next),
        out_shape=tuple(outs),
        scratch_shapes=[pltpu.VMEM((R, 128), jnp.uint32),
                        pltpu.VMEM((R, 128), jnp.int32)],
    )(*args)


def _make_sc_gather(k_pad, kw, with_d):
    mesh = plsc.VectorSubcoreMesh(core_axis_name="c", subcore_axis_name="s")
    outs = [jax.ShapeDtypeStruct((k_pad, 128), jnp.float32)]
    scratch = [
        pltpu.VMEM((kw,), jnp.int32),
        pltpu.VMEM((kw, 128), jnp.float32),
        pltpu.SemaphoreType.DMA,
    ]
    if with_d:
        outs.append(jax.ShapeDtypeStruct((k_pad,), jnp.float32))
        scratch.append(pltpu.VMEM((kw,), jnp.float32))

    if with_d:
        @functools.partial(
            pl.kernel, mesh=mesh, out_type=tuple(outs),
            scratch_types=scratch)
        def sc_gather(z_hbm, idx_hbm, d_hbm, out_hbm, dout_hbm,
                      idx_v, rows_v, sem, dv_v):
            wid = lax.axis_index("s") * _NC + lax.axis_index("c")
            base = wid * kw
            pltpu.sync_copy(idx_hbm.at[pl.ds(base, kw)], idx_v)
            pltpu.async_copy(z_hbm.at[idx_v], rows_v, sem).wait()
            pltpu.async_copy(d_hbm.at[idx_v], dv_v, sem).wait()
            pltpu.sync_copy(rows_v, out_hbm.at[pl.ds(base, kw)])
            pltpu.sync_copy(dv_v, dout_hbm.at[pl.ds(base, kw)])
    else:
        @functools.partial(
            pl.kernel, mesh=mesh, out_type=outs[0],
            scratch_types=scratch)
        def sc_gather(z_hbm, idx_hbm, out_hbm, idx_v, rows_v, sem):
            wid = lax.axis_index("s") * _NC + lax.axis_index("c")
            base = wid * kw
            pltpu.sync_copy(idx_hbm.at[pl.ds(base, kw)], idx_v)
            pltpu.async_copy(z_hbm.at[idx_v], rows_v, sem).wait()
            pltpu.sync_copy(rows_v, out_hbm.at[pl.ds(base, kw)])

    return sc_gather


def kernel(x, edge_index, batch, pool_w0, pool_w1, pool_w2,
           proj_W0, proj_b0, proj_W1, proj_b1, proj_W2, proj_b2):
    pws = [pool_w0, pool_w1, pool_w2]
    Ws = [proj_W0, proj_W1, proj_W2]
    bs = [proj_b0, proj_b1, proj_b2]
    norms = [jnp.linalg.norm(pw) for pw in pws]

    feats = [x]
    cur = x
    n0, P0, R0 = _LEVELS[0][0], _LEVELS[0][1], _LEVELS[0][2]
    dot_lane = _a_level(x, pws[0], n0, P0, R0)
    raw_flat = dot_lane.reshape(P0)[:n0]

    for i, (n, P, R, k, kw, k_pad, kpR) in enumerate(_LEVELS):
        s_flat = jnp.tanh(raw_flat / norms[i])
        s_full = jnp.broadcast_to(s_flat[:, None], (n, 128))
        s_lane = jnp.concatenate(
            [s_flat, jnp.zeros((P - n,), jnp.float32)]).reshape(R, 128)
        pw_next = pws[i + 1] if i < 2 else None
        res = _b_level(cur, s_full, s_lane, Ws[i], bs[i], pw_next,
                       n, P, R, kpR)
        if pw_next is not None:
            z, dn_lane, perm_mat = res
            d_flat = dn_lane.reshape(P)
            pooled_pad, d_next = _make_sc_gather(k_pad, kw, True)(
                z, perm_mat.reshape(k_pad), d_flat)
            raw_flat = d_next[:k]
        else:
            z, perm_mat = res
            pooled_pad = _make_sc_gather(k_pad, kw, False)(
                z, perm_mat.reshape(k_pad))
        pooled = pooled_pad[:k]
        feats.append(pooled)
        cur = pooled
    return tuple(feats)


# speedup vs baseline: 2.3569x; 2.3569x over previous
import functools

import jax
import jax.numpy as jnp
from jax import lax
from jax.experimental import pallas as pl
from jax.experimental.pallas import tpu as pltpu
from jax.experimental.pallas import tpu_sc as plsc

_NW = 32
_NC = 2

_LEVELS = [
    (10000, 16384, 128, 7000, 224, 7168, 56),
    (7000, 8192, 64, 3500, 112, 3584, 28),
    (3500, 4096, 32, 1050, 48, 1536, 12),
]


def _iota_e(R):
    return (lax.broadcasted_iota(jnp.int32, (R, 128), 0) * 128 +
            lax.broadcasted_iota(jnp.int32, (R, 128), 1))


def _bitonic(key, idx, P, R):
    e = _iota_e(R)
    LOG = P.bit_length() - 1
    for jj in range(1, LOG + 1):
        size = 1 << jj
        for kk in [1 << t for t in range(jj - 1, -1, -1)]:
            if kk >= 128:
                m = kk // 128
                def rowswap(a, m=m):
                    a4 = a.reshape(R // (2 * m), 2, m, 128)
                    return jnp.concatenate([a4[:, 1], a4[:, 0]],
                                           axis=1).reshape(R, 128)
                pk, pi = rowswap(key), rowswap(idx)
            else:
                mb = (e & kk) != 0
                pk = jnp.where(mb, jnp.roll(key, kk, axis=1),
                               jnp.roll(key, -kk, axis=1))
                pi = jnp.where(mb, jnp.roll(idx, kk, axis=1),
                               jnp.roll(idx, -kk, axis=1))
            upper = (e & kk) != 0
            desc = (e & size) != 0
            p_less = (pk < key) | ((pk == key) & (pi < idx))
            take_p = p_less ^ upper ^ desc
            key = jnp.where(take_p, pk, key)
            idx = jnp.where(take_p, pi, idx)
    return key, idx


def _diag_extract(M, n, P, R):
    Mp = jnp.concatenate([M, jnp.zeros((P - n, 128), jnp.float32)], axis=0)
    eye = (lax.broadcasted_iota(jnp.int32, (128, 128), 0) ==
           lax.broadcasted_iota(jnp.int32, (128, 128), 1)
           ).astype(jnp.float32)
    return jnp.sum(Mp.reshape(R, 128, 128) * eye[None], axis=1)


def _keys_from_scores(sc, n, R):
    sc = sc + 0.0
    b = lax.bitcast_convert_type(sc, jnp.uint32)
    sign = b >> 31
    u_asc = jnp.where(sign == jnp.uint32(1), ~b, b | jnp.uint32(0x80000000))
    key = ~u_asc
    return jnp.where(_iota_e(R) < n, key, jnp.uint32(0xFFFFFFFF))


def _make_a_body(n, P, R):
    def body(cur_ref, w2_ref, dot_ref):
        M = jnp.dot(cur_ref[...], w2_ref[...],
                    preferred_element_type=jnp.float32)
        dot_ref[...] = _diag_extract(M, n, P, R)
    return body


def _a_level(cur, pw, n, P, R):
    W2 = jnp.broadcast_to(pw[:, None], (128, 128))
    return pl.pallas_call(
        _make_a_body(n, P, R),
        out_shape=jax.ShapeDtypeStruct((R, 128), jnp.float32),
    )(cur, W2)


def _make_b_body(n, P, R, kpR, with_next):
    def body(*refs):
        if with_next:
            (scaled_ref, slane_ref, wt_ref, b_ref, w2n_ref,
             z_ref, dn_ref, perm_ref) = refs
        else:
            (scaled_ref, slane_ref, wt_ref, b_ref,
             z_ref, perm_ref) = refs
        z = jnp.dot(scaled_ref[...], wt_ref[...],
                    preferred_element_type=jnp.float32) + b_ref[...]
        z_ref[...] = z
        if with_next:
            Dn = jnp.dot(z, w2n_ref[...], preferred_element_type=jnp.float32)
            dn_ref[...] = _diag_extract(Dn, n, P, R)
        key = _keys_from_scores(slane_ref[...], n, R)
        _, sidx = _bitonic(key, _iota_e(R), P, R)
        perm_ref[...] = sidx[:kpR]
    return body


def _b_level(scaled, s_lane, W, b, pw_next, n, P, R, kpR):
    with_next = pw_next is not None
    outs = [jax.ShapeDtypeStruct((n, 128), jnp.float32)]
    if with_next:
        outs.append(jax.ShapeDtypeStruct((R, 128), jnp.float32))
    outs.append(jax.ShapeDtypeStruct((kpR, 128), jnp.int32))
    args = [scaled, s_lane, W.T, b[None, :]]
    if with_next:
        args.append(jnp.broadcast_to(pw_next[:, None], (128, 128)))
    return pl.pallas_call(
        _make_b_body(n, P, R, kpR, with_next),
        out_shape=tuple(outs),
    )(*args)


def _make_sc_gather(k_pad, kw, with_d):
    mesh = plsc.VectorSubcoreMesh(core_axis_name="c", subcore_axis_name="s")
    outs = [jax.ShapeDtypeStruct((k_pad, 128), jnp.float32)]
    scratch = [
        pltpu.VMEM((kw,), jnp.int32),
        pltpu.VMEM((kw, 128), jnp.float32),
        pltpu.SemaphoreType.DMA,
    ]
    if with_d:
        outs.append(jax.ShapeDtypeStruct((k_pad,), jnp.float32))
        scratch.append(pltpu.VMEM((kw,), jnp.float32))

    if with_d:
        @functools.partial(
            pl.kernel, mesh=mesh, out_type=tuple(outs),
            scratch_types=scratch)
        def sc_gather(z_hbm, idx_hbm, d_hbm, out_hbm, dout_hbm,
                      idx_v, rows_v, sem, dv_v):
            wid = lax.axis_index("s") * _NC + lax.axis_index("c")
            base = wid * kw
            pltpu.sync_copy(idx_hbm.at[pl.ds(base, kw)], idx_v)
            pltpu.async_copy(z_hbm.at[idx_v], rows_v, sem).wait()
            pltpu.async_copy(d_hbm.at[idx_v], dv_v, sem).wait()
            pltpu.sync_copy(rows_v, out_hbm.at[pl.ds(base, kw)])
            pltpu.sync_copy(dv_v, dout_hbm.at[pl.ds(base, kw)])
    else:
        @functools.partial(
            pl.kernel, mesh=mesh, out_type=outs[0],
            scratch_types=scratch)
        def sc_gather(z_hbm, idx_hbm, out_hbm, idx_v, rows_v, sem):
            wid = lax.axis_index("s") * _NC + lax.axis_index("c")
            base = wid * kw
            pltpu.sync_copy(idx_hbm.at[pl.ds(base, kw)], idx_v)
            pltpu.async_copy(z_hbm.at[idx_v], rows_v, sem).wait()
            pltpu.sync_copy(rows_v, out_hbm.at[pl.ds(base, kw)])

    return sc_gather


def kernel(x, edge_index, batch, pool_w0, pool_w1, pool_w2,
           proj_W0, proj_b0, proj_W1, proj_b1, proj_W2, proj_b2):
    pws = [pool_w0, pool_w1, pool_w2]
    Ws = [proj_W0, proj_W1, proj_W2]
    bs = [proj_b0, proj_b1, proj_b2]
    norms = [jnp.linalg.norm(pw) for pw in pws]

    feats = [x]
    cur = x
    n0, P0, R0 = _LEVELS[0][0], _LEVELS[0][1], _LEVELS[0][2]
    dot_lane = _a_level(x, pws[0], n0, P0, R0)
    raw_flat = dot_lane.reshape(P0)[:n0]

    for i, (n, P, R, k, kw, k_pad, kpR) in enumerate(_LEVELS):
        s_flat = jnp.tanh(raw_flat / norms[i])
        scaled = cur * s_flat[:, None]
        s_lane = jnp.concatenate(
            [s_flat, jnp.zeros((P - n,), jnp.float32)]).reshape(R, 128)
        pw_next = pws[i + 1] if i < 2 else None
        res = _b_level(scaled, s_lane, Ws[i], bs[i], pw_next,
                       n, P, R, kpR)
        if pw_next is not None:
            z, dn_lane, perm_mat = res
            d_flat = dn_lane.reshape(P)
            pooled_pad, d_next = _make_sc_gather(k_pad, kw, True)(
                z, perm_mat.reshape(k_pad), d_flat)
            raw_flat = d_next[:k]
        else:
            z, perm_mat = res
            pooled_pad = _make_sc_gather(k_pad, kw, False)(
                z, perm_mat.reshape(k_pad))
        pooled = pooled_pad[:k]
        feats.append(pooled)
        cur = pooled
    return tuple(feats)
